# Initial kernel scaffold; baseline (speedup 1.0000x reference)
import functools

import jax
import jax.numpy as jnp
from jax import lax
from jax.experimental import pallas as pl
from jax.experimental.pallas import tpu as pltpu
from jax.experimental.pallas import tpu_sc as plsc

_B, _N, _NT = 8, 1024, 1024
_D = 131
_DP = 144
_K = 3
_E = _K * _N
_STEPS = 4
_NC, _NS = 2, 16
_RBLK = 256
_NBLK = 512
_EPS = 1e-5
_F32 = jnp.float32
_HI = lax.Precision.HIGHEST


def _knn_body(xr_ref, xa_ref, out_ref):
    Xr = xr_ref[0]
    Xa = xa_ref[0]
    sq = Xa * Xa
    ones = jnp.ones((8, 128), _F32)
    x2 = lax.dot_general(ones, sq, (((1,), (1,)), ((), ())),
                         precision=_HI, preferred_element_type=_F32)[0:1]
    G = lax.dot_general(Xr, Xa, (((1,), (1,)), ((), ())),
                        precision=_HI, preferred_element_type=_F32)
    d = x2 - 2.0 * G
    r = pl.program_id(1)
    rows = lax.broadcasted_iota(jnp.int32, (_RBLK, _N), 0) + r * _RBLK
    lanes = lax.broadcasted_iota(jnp.int32, (_RBLK, _N), 1)
    d = jnp.where(rows == lanes, jnp.inf, d)
    acc = jnp.zeros((_RBLK, 128), jnp.int32)
    tl = lax.broadcasted_iota(jnp.int32, (_RBLK, 128), 1)
    for t in range(_K):
        m = jnp.min(d, axis=1, keepdims=True)
        idx = jnp.min(jnp.where(d == m, lanes, _N), axis=1, keepdims=True)
        d = jnp.where(lanes == idx, jnp.inf, d)
        acc = jnp.where(tl == t, idx, acc)
    out_ref[0] = acc


def _knn(xcp):
    return pl.pallas_call(
        _knn_body,
        grid=(_B, _N // _RBLK),
        in_specs=[pl.BlockSpec((1, _RBLK, 128), lambda b, r: (b, r, 0)),
                  pl.BlockSpec((1, _N, 128), lambda b, r: (b, 0, 0))],
        out_specs=pl.BlockSpec((1, _RBLK, 128), lambda b, r: (b, r, 0)),
        out_shape=jax.ShapeDtypeStruct((_B, _N, 128), jnp.int32),
    )(xcp, xcp)


def _enc_body(in8_ref, xcp_ref, w0, b0, w1, b1, w2, b2,
              wxa, wha, wxb, whb, lat_ref, ab_ref):
    x = in8_ref[0]
    h = jnp.maximum(jnp.dot(x, w0[...], preferred_element_type=_F32) + b0[...], 0.0)
    h = jnp.maximum(jnp.dot(h, w1[...], preferred_element_type=_F32) + b1[...], 0.0)
    latv = jnp.dot(h, w2[...], preferred_element_type=_F32) + b2[...]
    lat_ref[0] = latv
    xcp = xcp_ref[0]
    ab_ref[0, :, 0, :] = (jnp.dot(xcp, wxa[...], preferred_element_type=_F32)
                          + jnp.dot(latv, wha[...], preferred_element_type=_F32))
    ab_ref[0, :, 1, :] = (jnp.dot(xcp, wxb[...], preferred_element_type=_F32)
                          + jnp.dot(latv, whb[...], preferred_element_type=_F32))


def _enc(in8, xcp, w0, b0, w1, b1, w2, b2, wxa, wha, wxb, whb):
    wspec = pl.BlockSpec(lambda b, n: (0, 0))
    return pl.pallas_call(
        _enc_body,
        grid=(_B, _N // _NBLK),
        in_specs=[pl.BlockSpec((1, _NBLK, 8), lambda b, n: (b, n, 0)),
                  pl.BlockSpec((1, _NBLK, 128), lambda b, n: (b, n, 0)),
                  wspec, wspec, wspec, wspec, wspec, wspec,
                  wspec, wspec, wspec, wspec],
        out_specs=[pl.BlockSpec((1, _NBLK, 128), lambda b, n: (b, n, 0)),
                   pl.BlockSpec((1, _NBLK, 2, _DP), lambda b, n: (b, n, 0, 0))],
        out_shape=[jax.ShapeDtypeStruct((_B, _N, 128), _F32),
                   jax.ShapeDtypeStruct((_B, _N, 2, _DP), _F32)],
    )(in8, xcp, w0, b0, w1, b1, w2, b2, wxa, wha, wxb, whb)


def _msg_body(ab_ref, gab_ref, mb_ref, g1_ref, b1_ref, m1_ref, in2_ref):
    A = ab_ref[0, :, 0, :]
    Bm = ab_ref[0, :, 1, :]
    mb = mb_ref[...]
    g1 = g1_ref[...]
    b1 = b1_ref[...]
    lanes = lax.broadcasted_iota(jnp.int32, (1, _DP), 1)
    mask = lanes < _D

    def ln(x):
        mu = jnp.sum(x, axis=1, keepdims=True) * (1.0 / _D)
        dm = jnp.where(mask, x - mu, 0.0)
        var = jnp.sum(dm * dm, axis=1, keepdims=True) * (1.0 / _D)
        return dm * lax.rsqrt(var + _EPS) * g1 + b1

    acc = jnp.zeros((_NBLK, _DP), _F32)
    for k in range(_K):
        GA = gab_ref[0, k, :, 0, :]
        GB = gab_ref[0, k, :, 1, :]
        m1_ref[0, k] = ln(A + GB + mb)
        acc = acc + ln(GA + Bm + mb)
    in2_ref[0] = acc


def _msg(AB, GAB, mbp, g1p, b1p):
    wspec = pl.BlockSpec(lambda b, n: (0, 0))
    return pl.pallas_call(
        _msg_body,
        grid=(_B, _N // _NBLK),
        in_specs=[pl.BlockSpec((1, _NBLK, 2, _DP), lambda b, n: (b, n, 0, 0)),
                  pl.BlockSpec((1, _K, _NBLK, 2, _DP), lambda b, n: (b, 0, n, 0, 0)),
                  wspec, wspec, wspec],
        out_specs=[pl.BlockSpec((1, _K, _NBLK, _DP), lambda b, n: (b, 0, n, 0)),
                   pl.BlockSpec((1, _NBLK, _DP), lambda b, n: (b, n, 0))],
        out_shape=[jax.ShapeDtypeStruct((_B, _K, _N, _DP), _F32),
                   jax.ShapeDtypeStruct((_B, _N, _DP), _F32)],
    )(AB, GAB, mbp, g1p, b1p)


def _upd_body(xcp_ref, lat_ref, inb_ref, wnx, wnh, wni, nb, g2, b2,
              wxa, wha, wxb, whb, lat2_ref, ab_ref):
    xcp = xcp_ref[0]
    o = (jnp.dot(xcp, wnx[...], preferred_element_type=_F32)
         + jnp.dot(lat_ref[0], wnh[...], preferred_element_type=_F32)
         + jnp.dot(inb_ref[0], wni[...], preferred_element_type=_F32)
         + nb[...])
    mu = jnp.mean(o, axis=1, keepdims=True)
    dm = o - mu
    var = jnp.mean(dm * dm, axis=1, keepdims=True)
    lat2 = dm * lax.rsqrt(var + _EPS) * g2[...] + b2[...]
    lat2_ref[0] = lat2
    ab_ref[0, :, 0, :] = (jnp.dot(xcp, wxa[...], preferred_element_type=_F32)
                          + jnp.dot(lat2, wha[...], preferred_element_type=_F32))
    ab_ref[0, :, 1, :] = (jnp.dot(xcp, wxb[...], preferred_element_type=_F32)
                          + jnp.dot(lat2, whb[...], preferred_element_type=_F32))


def _upd(xcp, lat, inbox, wnx, wnh, wni, nbp, g2p, b2p, wxa, wha, wxb, whb):
    wspec = pl.BlockSpec(lambda b, n: (0, 0))
    return pl.pallas_call(
        _upd_body,
        grid=(_B, _N // _NBLK),
        in_specs=[pl.BlockSpec((1, _NBLK, 128), lambda b, n: (b, n, 0)),
                  pl.BlockSpec((1, _NBLK, 128), lambda b, n: (b, n, 0)),
                  pl.BlockSpec((1, _NBLK, _DP), lambda b, n: (b, n, 0)),
                  wspec, wspec, wspec, wspec, wspec, wspec,
                  wspec, wspec, wspec, wspec],
        out_specs=[pl.BlockSpec((1, _NBLK, 128), lambda b, n: (b, n, 0)),
                   pl.BlockSpec((1, _NBLK, 2, _DP), lambda b, n: (b, n, 0, 0))],
        out_shape=[jax.ShapeDtypeStruct((_B, _N, 128), _F32),
                   jax.ShapeDtypeStruct((_B, _N, 2, _DP), _F32)],
    )(xcp, lat, inbox, wnx, wnh, wni, nbp, g2p, b2p, wxa, wha, wxb, whb)


def _dec_body(xt_ref, xc_ref, lat_ref, ls_ref, w0z, w0x, b0, w1, b1, w2p, b2p,
              out_ref):
    XT = xt_ref[0]
    XC = xc_ref[0]
    L = lat_ref[0]
    strength = jnp.exp(ls_ref[0, 0])
    sq = XC * XC
    ones = jnp.ones((8, 128), _F32)
    x2 = lax.dot_general(ones, sq, (((1,), (1,)), ((), ())),
                         precision=_HI, preferred_element_type=_F32)[0:1]
    P = lax.dot_general(XT, XC, (((1,), (1,)), ((), ())),
                        precision=_HI, preferred_element_type=_F32)
    logits = -strength * (x2 - 2.0 * P)
    m = jnp.max(logits, axis=1, keepdims=True)
    e = jnp.exp(logits - m)
    s = jnp.sum(e, axis=1, keepdims=True)
    z = jnp.dot(e / s, L, preferred_element_type=_F32)
    h = jnp.maximum(jnp.dot(z, w0z[...], preferred_element_type=_F32)
                    + jnp.dot(XT, w0x[...], preferred_element_type=_F32)
                    + b0[...], 0.0)
    h = jnp.maximum(jnp.dot(h, w1[...], preferred_element_type=_F32) + b1[...], 0.0)
    out_ref[0] = jnp.dot(h, w2p[...], preferred_element_type=_F32) + b2p[...]


def _dec(xtp, xcp, lat, ls, w0z, w0x, bd0, w1, bd1, w2p, bd2):
    wspec = pl.BlockSpec(lambda b, r: (0, 0))
    return pl.pallas_call(
        _dec_body,
        grid=(_B, _NT // _RBLK),
        in_specs=[pl.BlockSpec((1, _RBLK, 128), lambda b, r: (b, r, 0)),
                  pl.BlockSpec((1, _N, 128), lambda b, r: (b, 0, 0)),
                  pl.BlockSpec((1, _N, 128), lambda b, r: (b, 0, 0)),
                  pl.BlockSpec((1, 1), lambda b, r: (0, 0),
                               memory_space=pltpu.SMEM),
                  wspec, wspec, wspec, wspec, wspec, wspec, wspec],
        out_specs=pl.BlockSpec((1, _RBLK, 128), lambda b, r: (b, r, 0)),
        out_shape=jax.ShapeDtypeStruct((_B, _NT, 128), _F32),
    )(xtp, xcp, lat, ls, w0z, w0x, bd0, w1, bd1, w2p, bd2)


_CS = 96


def _sc_gather(ab2, nbr_off):
    per = _E // _NS
    mesh = plsc.VectorSubcoreMesh(core_axis_name="c", subcore_axis_name="s")

    @functools.partial(
        pl.kernel,
        out_type=jax.ShapeDtypeStruct((_B * _E, 2 * _DP), _F32),
        mesh=mesh,
        scratch_types=[pltpu.VMEM((2, _CS), jnp.int32),
                       pltpu.VMEM((per, 2 * _DP), _F32),
                       pltpu.SemaphoreType.DMA])
    def gk(ab_hbm, idx_hbm, out_hbm, idx_v, rows_v, sem):
        cid = lax.axis_index("c")
        sid = lax.axis_index("s")
        for j in range(_B // _NC):
            b = j * _NC + cid
            base = b * _E + sid * per
            for c in range(2):
                pltpu.sync_copy(idx_hbm.at[pl.ds(base + c * _CS, _CS)],
                                idx_v.at[c])
                pltpu.async_copy(ab_hbm.at[idx_v.at[c]],
                                 rows_v.at[pl.ds(c * _CS, _CS)], sem).wait()
            pltpu.sync_copy(rows_v, out_hbm.at[pl.ds(base, per)])

    return gk(ab2, nbr_off)


def _sc_scatter(m1f, nbr_loc, in2f):
    per = _E // _NS
    npt = _N // _NS
    mesh = plsc.VectorSubcoreMesh(core_axis_name="c", subcore_axis_name="s")

    @functools.partial(
        pl.kernel,
        out_type=jax.ShapeDtypeStruct((_B * _N, _DP), _F32),
        mesh=mesh,
        scratch_types=[pltpu.VMEM_SHARED((_N, _DP), _F32),
                       pltpu.VMEM((2, _CS), jnp.int32),
                       pltpu.VMEM((per, _DP), _F32),
                       pltpu.SemaphoreType.DMA])
    def sk(m1_hbm, idx_hbm, in2_hbm, out_hbm, acc, idx_v, rows_v, sem):
        cid = lax.axis_index("c")
        sid = lax.axis_index("s")
        for j in range(_B // _NC):
            b = j * _NC + cid
            ebase = b * _E + sid * per
            nbase = b * _N + sid * npt
            pltpu.sync_copy(in2_hbm.at[pl.ds(nbase, npt)],
                            acc.at[pl.ds(sid * npt, npt)])
            pltpu.sync_copy(m1_hbm.at[pl.ds(ebase, per)], rows_v)
            for c in range(2):
                pltpu.sync_copy(idx_hbm.at[pl.ds(ebase + c * _CS, _CS)],
                                idx_v.at[c])
            plsc.subcore_barrier()
            for c in range(2):
                pltpu.sync_copy(rows_v.at[pl.ds(c * _CS, _CS)],
                                acc.at[idx_v.at[c]], add=True)
            plsc.subcore_barrier()
            pltpu.sync_copy(acc.at[pl.ds(sid * npt, npt)],
                            out_hbm.at[pl.ds(nbase, npt)])

    return sk(m1f, nbr_loc, in2f)


def kernel(xc, yc, xt, enc_W0, enc_b0, enc_W1, enc_b1, enc_W2, enc_b2,
           dec_W0, dec_b0, dec_W1, dec_b1, dec_W2, dec_b2,
           msg_W, msg_b, node_W, node_b, ln1_g, ln1_b, ln2_g, ln2_b,
           log_strength):
    z128 = jnp.zeros((128, 128), _F32)
    zdp = jnp.zeros((128, _DP), _F32)

    xcp = jnp.pad(xc, ((0, 0), (0, 0), (0, 125)))
    xtp = jnp.pad(xt, ((0, 0), (0, 0), (0, 125)))
    in8 = jnp.pad(jnp.concatenate((xc, yc), -1), ((0, 0), (0, 0), (0, 2)))

    Ws, Wr = msg_W[:_D], msg_W[_D:]
    wxa = zdp.at[:3, :_D].set(Ws[:3])
    wha = zdp.at[:, :_D].set(Ws[3:])
    wxb = zdp.at[:3, :_D].set(Wr[:3])
    whb = zdp.at[:, :_D].set(Wr[3:])
    mbp = jnp.zeros((1, _DP), _F32).at[0, :_D].set(msg_b)
    g1p = jnp.zeros((1, _DP), _F32).at[0, :_D].set(ln1_g)
    b1p = jnp.zeros((1, _DP), _F32).at[0, :_D].set(ln1_b)

    wnx = z128.at[:3].set(node_W[:3])
    wnh = node_W[3:_D]
    wni = jnp.zeros((_DP, 128), _F32).at[:_D].set(node_W[_D:])
    nbp = node_b.reshape(1, 128)
    g2p = ln2_g.reshape(1, 128)
    b2p = ln2_b.reshape(1, 128)

    w0e = jnp.zeros((8, 128), _F32).at[:6].set(enc_W0)
    be0 = enc_b0.reshape(1, 128)
    be1 = enc_b1.reshape(1, 128)
    be2 = enc_b2.reshape(1, 128)

    w0z = dec_W0[:128]
    w0x = z128.at[:3].set(dec_W0[128:_D])
    bd0 = dec_b0.reshape(1, 128)
    bd1 = dec_b1.reshape(1, 128)
    w2p = jnp.zeros((128, 128), _F32).at[:, :3].set(dec_W2)
    bd2 = jnp.zeros((1, 128), _F32).at[0, :3].set(dec_b2)
    ls = log_strength.reshape(1, 1)

    nbrcols = _knn(xcp)
    nbr_t = jnp.transpose(nbrcols[:, :, :_K], (0, 2, 1))
    nbr_loc = nbr_t.reshape(_B * _E)
    nbr_off = (nbr_t.reshape(_B, _E)
               + (jnp.arange(_B, dtype=jnp.int32) * _N)[:, None]).reshape(-1)

    lat, AB = _enc(in8, xcp, w0e, be0, enc_W1, be1, enc_W2, be2,
                   wxa, wha, wxb, whb)
    for _ in range(_STEPS):
        GABf = _sc_gather(AB.reshape(_B * _N, 2 * _DP), nbr_off)
        GAB = GABf.reshape(_B, _K, _N, 2, _DP)
        M1, in2 = _msg(AB, GAB, mbp, g1p, b1p)
        inboxf = _sc_scatter(M1.reshape(_B * _E, _DP), nbr_loc,
                             in2.reshape(_B * _N, _DP))
        lat, AB = _upd(xcp, lat, inboxf.reshape(_B, _N, _DP),
                       wnx, wnh, wni, nbp, g2p, b2p, wxa, wha, wxb, whb)

    y = _dec(xtp, xcp, lat, ls, w0z, w0x, bd0, dec_W1, bd1, w2p, bd2)
    return y[:, :, :3]


# speedup vs baseline: 377.6666x; 377.6666x over previous
import functools

import jax
import jax.numpy as jnp
from jax import lax
from jax.experimental import pallas as pl
from jax.experimental.pallas import tpu as pltpu
from jax.experimental.pallas import tpu_sc as plsc

_B, _N, _NT = 8, 1024, 1024
_D = 131
_DP = 144
_K = 3
_E = _K * _N
_STEPS = 4
_NC, _NS = 2, 16
_RBLK = 256
_NBLK = 512
_EPS = 1e-5
_F32 = jnp.float32
_HI = lax.Precision.HIGHEST


def _fs(*shape):
    return pl.BlockSpec(shape, lambda *_: (0,) * len(shape))

def _knn_body(xr_ref, xa_ref, out_ref):
    Xr = xr_ref[0]
    Xa = xa_ref[0]
    sq = Xa * Xa
    ones = jnp.ones((8, 128), _F32)
    x2 = lax.dot_general(ones, sq, (((1,), (1,)), ((), ())),
                         precision=_HI, preferred_element_type=_F32)[0:1]
    G = lax.dot_general(Xr, Xa, (((1,), (1,)), ((), ())),
                        precision=_HI, preferred_element_type=_F32)
    d = x2 - 2.0 * G
    r = pl.program_id(1)
    rows = lax.broadcasted_iota(jnp.int32, (_RBLK, _N), 0) + r * _RBLK
    lanes = lax.broadcasted_iota(jnp.int32, (_RBLK, _N), 1)
    d = jnp.where(rows == lanes, jnp.inf, d)
    acc = jnp.zeros((_RBLK, 128), jnp.int32)
    tl = lax.broadcasted_iota(jnp.int32, (_RBLK, 128), 1)
    for t in range(_K):
        m = jnp.min(d, axis=1, keepdims=True)
        idx = jnp.min(jnp.where(d == m, lanes, _N), axis=1, keepdims=True)
        d = jnp.where(lanes == idx, jnp.inf, d)
        acc = jnp.where(tl == t, idx, acc)
    out_ref[0] = acc


def _knn(xcp):
    return pl.pallas_call(
        _knn_body,
        grid=(_B, _N // _RBLK),
        in_specs=[pl.BlockSpec((1, _RBLK, 128), lambda b, r: (b, r, 0)),
                  pl.BlockSpec((1, _N, 128), lambda b, r: (b, 0, 0))],
        out_specs=pl.BlockSpec((1, _RBLK, 128), lambda b, r: (b, r, 0)),
        out_shape=jax.ShapeDtypeStruct((_B, _N, 128), jnp.int32),
    )(xcp, xcp)


def _enc_body(in8_ref, xcp_ref, w0, b0, w1, b1, w2, b2,
              wxa, wha, wxb, whb, lat_ref, ab_ref):
    x = in8_ref[0]
    h = jnp.maximum(jnp.dot(x, w0[...], preferred_element_type=_F32) + b0[...], 0.0)
    h = jnp.maximum(jnp.dot(h, w1[...], preferred_element_type=_F32) + b1[...], 0.0)
    latv = jnp.dot(h, w2[...], preferred_element_type=_F32) + b2[...]
    lat_ref[0] = latv
    xcp = xcp_ref[0]
    ab_ref[0, :, 0, :] = (jnp.dot(xcp, wxa[...], preferred_element_type=_F32)
                          + jnp.dot(latv, wha[...], preferred_element_type=_F32))
    ab_ref[0, :, 1, :] = (jnp.dot(xcp, wxb[...], preferred_element_type=_F32)
                          + jnp.dot(latv, whb[...], preferred_element_type=_F32))


def _enc(in8, xcp, w0, b0, w1, b1, w2, b2, wxa, wha, wxb, whb):
    return pl.pallas_call(
        _enc_body,
        grid=(_B, _N // _NBLK),
        in_specs=[pl.BlockSpec((1, _NBLK, 8), lambda b, n: (b, n, 0)),
                  pl.BlockSpec((1, _NBLK, 128), lambda b, n: (b, n, 0)),
                  _fs(8, 128), _fs(1, 128), _fs(128, 128), _fs(1, 128),
                  _fs(128, 128), _fs(1, 128),
                  _fs(128, _DP), _fs(128, _DP), _fs(128, _DP), _fs(128, _DP)],
        out_specs=[pl.BlockSpec((1, _NBLK, 128), lambda b, n: (b, n, 0)),
                   pl.BlockSpec((1, _NBLK, 2, _DP), lambda b, n: (b, n, 0, 0))],
        out_shape=[jax.ShapeDtypeStruct((_B, _N, 128), _F32),
                   jax.ShapeDtypeStruct((_B, _N, 2, _DP), _F32)],
    )(in8, xcp, w0, b0, w1, b1, w2, b2, wxa, wha, wxb, whb)


def _msg_body(ab_ref, gab_ref, mb_ref, g1_ref, b1_ref, m1_ref, in2_ref):
    A = ab_ref[0, :, 0, :]
    Bm = ab_ref[0, :, 1, :]
    mb = mb_ref[...]
    g1 = g1_ref[...]
    b1 = b1_ref[...]
    lanes = lax.broadcasted_iota(jnp.int32, (1, _DP), 1)
    mask = lanes < _D

    def ln(x):
        mu = jnp.sum(x, axis=1, keepdims=True) * (1.0 / _D)
        dm = jnp.where(mask, x - mu, 0.0)
        var = jnp.sum(dm * dm, axis=1, keepdims=True) * (1.0 / _D)
        return dm * lax.rsqrt(var + _EPS) * g1 + b1

    acc = jnp.zeros((_NBLK, _DP), _F32)
    for k in range(_K):
        GA = gab_ref[0, k, :, 0, :]
        GB = gab_ref[0, k, :, 1, :]
        m1_ref[0, k] = ln(A + GB + mb)
        acc = acc + ln(GA + Bm + mb)
    in2_ref[0] = acc


def _msg(AB, GAB, mbp, g1p, b1p):
    return pl.pallas_call(
        _msg_body,
        grid=(_B, _N // _NBLK),
        in_specs=[pl.BlockSpec((1, _NBLK, 2, _DP), lambda b, n: (b, n, 0, 0)),
                  pl.BlockSpec((1, _K, _NBLK, 2, _DP), lambda b, n: (b, 0, n, 0, 0)),
                  _fs(1, _DP), _fs(1, _DP), _fs(1, _DP)],
        out_specs=[pl.BlockSpec((1, _K, _NBLK, _DP), lambda b, n: (b, 0, n, 0)),
                   pl.BlockSpec((1, _NBLK, _DP), lambda b, n: (b, n, 0))],
        out_shape=[jax.ShapeDtypeStruct((_B, _K, _N, _DP), _F32),
                   jax.ShapeDtypeStruct((_B, _N, _DP), _F32)],
    )(AB, GAB, mbp, g1p, b1p)


def _upd_body(xcp_ref, lat_ref, inb_ref, wnx, wnh, wni, nb, g2, b2,
              wxa, wha, wxb, whb, lat2_ref, ab_ref):
    xcp = xcp_ref[0]
    o = (jnp.dot(xcp, wnx[...], preferred_element_type=_F32)
         + jnp.dot(lat_ref[0], wnh[...], preferred_element_type=_F32)
         + jnp.dot(inb_ref[0], wni[...], preferred_element_type=_F32)
         + nb[...])
    mu = jnp.mean(o, axis=1, keepdims=True)
    dm = o - mu
    var = jnp.mean(dm * dm, axis=1, keepdims=True)
    lat2 = dm * lax.rsqrt(var + _EPS) * g2[...] + b2[...]
    lat2_ref[0] = lat2
    ab_ref[0, :, 0, :] = (jnp.dot(xcp, wxa[...], preferred_element_type=_F32)
                          + jnp.dot(lat2, wha[...], preferred_element_type=_F32))
    ab_ref[0, :, 1, :] = (jnp.dot(xcp, wxb[...], preferred_element_type=_F32)
                          + jnp.dot(lat2, whb[...], preferred_element_type=_F32))


def _upd(xcp, lat, inbox, wnx, wnh, wni, nbp, g2p, b2p, wxa, wha, wxb, whb):
    return pl.pallas_call(
        _upd_body,
        grid=(_B, _N // _NBLK),
        in_specs=[pl.BlockSpec((1, _NBLK, 128), lambda b, n: (b, n, 0)),
                  pl.BlockSpec((1, _NBLK, 128), lambda b, n: (b, n, 0)),
                  pl.BlockSpec((1, _NBLK, _DP), lambda b, n: (b, n, 0)),
                  _fs(128, 128), _fs(128, 128), _fs(_DP, 128), _fs(1, 128),
                  _fs(1, 128), _fs(1, 128),
                  _fs(128, _DP), _fs(128, _DP), _fs(128, _DP), _fs(128, _DP)],
        out_specs=[pl.BlockSpec((1, _NBLK, 128), lambda b, n: (b, n, 0)),
                   pl.BlockSpec((1, _NBLK, 2, _DP), lambda b, n: (b, n, 0, 0))],
        out_shape=[jax.ShapeDtypeStruct((_B, _N, 128), _F32),
                   jax.ShapeDtypeStruct((_B, _N, 2, _DP), _F32)],
    )(xcp, lat, inbox, wnx, wnh, wni, nbp, g2p, b2p, wxa, wha, wxb, whb)


def _dec_body(xt_ref, xc_ref, lat_ref, ls_ref, w0z, w0x, b0, w1, b1, w2p, b2p,
              out_ref):
    XT = xt_ref[0]
    XC = xc_ref[0]
    L = lat_ref[0]
    strength = jnp.exp(ls_ref[0, 0])
    sq = XC * XC
    ones = jnp.ones((8, 128), _F32)
    x2 = lax.dot_general(ones, sq, (((1,), (1,)), ((), ())),
                         precision=_HI, preferred_element_type=_F32)[0:1]
    P = lax.dot_general(XT, XC, (((1,), (1,)), ((), ())),
                        precision=_HI, preferred_element_type=_F32)
    logits = -strength * (x2 - 2.0 * P)
    m = jnp.max(logits, axis=1, keepdims=True)
    e = jnp.exp(logits - m)
    s = jnp.sum(e, axis=1, keepdims=True)
    z = jnp.dot(e / s, L, preferred_element_type=_F32)
    h = jnp.maximum(jnp.dot(z, w0z[...], preferred_element_type=_F32)
                    + jnp.dot(XT, w0x[...], preferred_element_type=_F32)
                    + b0[...], 0.0)
    h = jnp.maximum(jnp.dot(h, w1[...], preferred_element_type=_F32) + b1[...], 0.0)
    out_ref[0] = jnp.dot(h, w2p[...], preferred_element_type=_F32) + b2p[...]


def _dec(xtp, xcp, lat, ls, w0z, w0x, bd0, w1, bd1, w2p, bd2):
    return pl.pallas_call(
        _dec_body,
        grid=(_B, _NT // _RBLK),
        in_specs=[pl.BlockSpec((1, _RBLK, 128), lambda b, r: (b, r, 0)),
                  pl.BlockSpec((1, _N, 128), lambda b, r: (b, 0, 0)),
                  pl.BlockSpec((1, _N, 128), lambda b, r: (b, 0, 0)),
                  pl.BlockSpec((1, 1), lambda b, r: (0, 0),
                               memory_space=pltpu.SMEM),
                  _fs(128, 128), _fs(128, 128), _fs(1, 128), _fs(128, 128),
                  _fs(1, 128), _fs(128, 128), _fs(1, 128)],
        out_specs=pl.BlockSpec((1, _RBLK, 128), lambda b, r: (b, r, 0)),
        out_shape=jax.ShapeDtypeStruct((_B, _NT, 128), _F32),
    )(xtp, xcp, lat, ls, w0z, w0x, bd0, w1, bd1, w2p, bd2)


_CS = 96


def _sc_gather(ab2, nbr_off):
    per = _E // _NS
    mesh = plsc.VectorSubcoreMesh(core_axis_name="c", subcore_axis_name="s")

    @functools.partial(
        pl.kernel,
        out_type=jax.ShapeDtypeStruct((_B * _E, 2 * _DP), _F32),
        mesh=mesh,
        compiler_params=pltpu.CompilerParams(use_tc_tiling_on_sc=False),
        scratch_types=[pltpu.VMEM((2, _CS), jnp.int32),
                       pltpu.VMEM((per, 2 * _DP), _F32),
                       pltpu.SemaphoreType.DMA])
    def gk(ab_hbm, idx_hbm, out_hbm, idx_v, rows_v, sem):
        cid = lax.axis_index("c")
        sid = lax.axis_index("s")
        for j in range(_B // _NC):
            b = j * _NC + cid
            base = b * _E + sid * per
            for c in range(2):
                pltpu.sync_copy(idx_hbm.at[pl.ds(base + c * _CS, _CS)],
                                idx_v.at[c])
                pltpu.async_copy(ab_hbm.at[idx_v.at[c]],
                                 rows_v.at[pl.ds(c * _CS, _CS)], sem).wait()
            pltpu.sync_copy(rows_v, out_hbm.at[pl.ds(base, per)])

    return gk(ab2, nbr_off)


def _sc_scatter(m1f, nbr_loc, in2f):
    per = _E // _NS
    npt = _N // _NS
    mesh = plsc.VectorSubcoreMesh(core_axis_name="c", subcore_axis_name="s")

    @functools.partial(
        pl.kernel,
        out_type=jax.ShapeDtypeStruct((_B * _N, _DP), _F32),
        mesh=mesh,
        compiler_params=pltpu.CompilerParams(use_tc_tiling_on_sc=False),
        scratch_types=[pltpu.VMEM_SHARED((_N, _DP), _F32),
                       pltpu.VMEM((2, _CS), jnp.int32),
                       pltpu.VMEM((per, _DP), _F32),
                       pltpu.SemaphoreType.DMA])
    def sk(m1_hbm, idx_hbm, in2_hbm, out_hbm, acc, idx_v, rows_v, sem):
        cid = lax.axis_index("c")
        sid = lax.axis_index("s")
        for j in range(_B // _NC):
            b = j * _NC + cid
            ebase = b * _E + sid * per
            nbase = b * _N + sid * npt
            pltpu.sync_copy(in2_hbm.at[pl.ds(nbase, npt)],
                            acc.at[pl.ds(sid * npt, npt)])
            pltpu.sync_copy(m1_hbm.at[pl.ds(ebase, per)], rows_v)
            for c in range(2):
                pltpu.sync_copy(idx_hbm.at[pl.ds(ebase + c * _CS, _CS)],
                                idx_v.at[c])
            plsc.subcore_barrier()
            for c in range(2):
                pltpu.sync_copy(rows_v.at[pl.ds(c * _CS, _CS)],
                                acc.at[idx_v.at[c]], add=True)
            plsc.subcore_barrier()
            pltpu.sync_copy(acc.at[pl.ds(sid * npt, npt)],
                            out_hbm.at[pl.ds(nbase, npt)])

    return sk(m1f, nbr_loc, in2f)


def kernel(xc, yc, xt, enc_W0, enc_b0, enc_W1, enc_b1, enc_W2, enc_b2,
           dec_W0, dec_b0, dec_W1, dec_b1, dec_W2, dec_b2,
           msg_W, msg_b, node_W, node_b, ln1_g, ln1_b, ln2_g, ln2_b,
           log_strength):
    z128 = jnp.zeros((128, 128), _F32)
    zdp = jnp.zeros((128, _DP), _F32)

    xcp = jnp.pad(xc, ((0, 0), (0, 0), (0, 125)))
    xtp = jnp.pad(xt, ((0, 0), (0, 0), (0, 125)))
    in8 = jnp.pad(jnp.concatenate((xc, yc), -1), ((0, 0), (0, 0), (0, 2)))

    Ws, Wr = msg_W[:_D], msg_W[_D:]
    wxa = zdp.at[:3, :_D].set(Ws[:3])
    wha = zdp.at[:, :_D].set(Ws[3:])
    wxb = zdp.at[:3, :_D].set(Wr[:3])
    whb = zdp.at[:, :_D].set(Wr[3:])
    mbp = jnp.zeros((1, _DP), _F32).at[0, :_D].set(msg_b)
    g1p = jnp.zeros((1, _DP), _F32).at[0, :_D].set(ln1_g)
    b1p = jnp.zeros((1, _DP), _F32).at[0, :_D].set(ln1_b)

    wnx = z128.at[:3].set(node_W[:3])
    wnh = node_W[3:_D]
    wni = jnp.zeros((_DP, 128), _F32).at[:_D].set(node_W[_D:])
    nbp = node_b.reshape(1, 128)
    g2p = ln2_g.reshape(1, 128)
    b2p = ln2_b.reshape(1, 128)

    w0e = jnp.zeros((8, 128), _F32).at[:6].set(enc_W0)
    be0 = enc_b0.reshape(1, 128)
    be1 = enc_b1.reshape(1, 128)
    be2 = enc_b2.reshape(1, 128)

    w0z = dec_W0[:128]
    w0x = z128.at[:3].set(dec_W0[128:_D])
    bd0 = dec_b0.reshape(1, 128)
    bd1 = dec_b1.reshape(1, 128)
    w2p = jnp.zeros((128, 128), _F32).at[:, :3].set(dec_W2)
    bd2 = jnp.zeros((1, 128), _F32).at[0, :3].set(dec_b2)
    ls = log_strength.reshape(1, 1)

    nbrcols = _knn(xcp)
    nbr_t = jnp.transpose(nbrcols[:, :, :_K], (0, 2, 1))
    nbr_loc = nbr_t.reshape(_B * _E)
    nbr_off = (nbr_t.reshape(_B, _E)
               + (jnp.arange(_B, dtype=jnp.int32) * _N)[:, None]).reshape(-1)

    lat, AB = _enc(in8, xcp, w0e, be0, enc_W1, be1, enc_W2, be2,
                   wxa, wha, wxb, whb)
    for _ in range(_STEPS):
        GABf = _sc_gather(AB.reshape(_B * _N, 2 * _DP), nbr_off)
        GAB = GABf.reshape(_B, _K, _N, 2, _DP)
        M1, in2 = _msg(AB, GAB, mbp, g1p, b1p)
        inboxf = _sc_scatter(M1.reshape(_B * _E, _DP), nbr_loc,
                             in2.reshape(_B * _N, _DP))
        lat, AB = _upd(xcp, lat, inboxf.reshape(_B, _N, _DP),
                       wnx, wnh, wni, nbp, g2p, b2p, wxa, wha, wxb, whb)

    y = _dec(xtp, xcp, lat, ls, w0z, w0x, bd0, dec_W1, bd1, w2p, bd2)
    return y[:, :, :3]
